# Initial kernel scaffold; baseline (speedup 1.0000x reference)
import jax
import jax.numpy as jnp
from jax.experimental import pallas as pl


def kernel(input_ids, weight):
    raise NotImplementedError("write your pallas kernel here")


# speedup vs baseline: 1.4468x; 1.4468x over previous
import jax
import jax.numpy as jnp
from jax import lax
from jax.experimental import pallas as pl
from jax.experimental.pallas import tpu as pltpu
from jax.experimental.pallas import tpu_sc as plsc

_K = 8


def _sc_lookup(weight, ids):
    n = ids.shape[0]
    _, d = weight.shape
    info = plsc.get_sparse_core_info()
    nw = info.num_cores * info.num_subcores
    per_w = n // nw
    nchunks = per_w // _K
    assert n % nw == 0 and per_w % _K == 0 and nchunks % 2 == 0

    mesh = plsc.VectorSubcoreMesh(
        core_axis_name="core", subcore_axis_name="subcore"
    )

    @pl.kernel(
        out_type=jax.ShapeDtypeStruct((n, d), weight.dtype),
        mesh=mesh,
        scratch_types=[
            pltpu.VMEM((per_w,), jnp.int32),
            pltpu.VMEM((_K, d), jnp.float32),
            pltpu.VMEM((_K, d), jnp.float32),
            pltpu.SemaphoreType.DMA,
            pltpu.SemaphoreType.DMA,
        ],
    )
    def lookup(w_hbm, i_hbm, o_hbm, idx_v, buf0, buf1, sem0, sem1):
        wid = lax.axis_index("subcore") * info.num_cores + lax.axis_index(
            "core"
        )
        base = wid * per_w
        pltpu.sync_copy(i_hbm.at[pl.ds(base, per_w)], idx_v)

        bufs = (buf0, buf1)
        sems = (sem0, sem1)

        def start_gather(c, b):
            pltpu.async_copy(
                w_hbm.at[idx_v.at[pl.ds(c * _K, _K)]], bufs[b], sems[b]
            )

        def wait_gather(b):
            pltpu.make_async_copy(
                w_hbm.at[pl.ds(0, _K)], bufs[b], sems[b]
            ).wait()

        def write_out(c, b):
            pltpu.sync_copy(bufs[b], o_hbm.at[pl.ds(base + c * _K, _K)])

        start_gather(0, 0)

        @pl.loop(0, nchunks - 2, step=2)
        def _(c0):
            for b in range(2):
                c = c0 + b
                start_gather(c + 1, 1 - b)
                wait_gather(b)
                write_out(c, b)

        start_gather(nchunks - 1, 1)
        wait_gather(0)
        write_out(nchunks - 2, 0)
        wait_gather(1)
        write_out(nchunks - 1, 1)

    return lookup(weight, ids)


def kernel(input_ids, weight):
    b, s = input_ids.shape
    d = weight.shape[1]
    out = _sc_lookup(weight, input_ids.reshape(-1))
    return out.reshape(b, s, d)
